# Initial kernel scaffold; baseline (speedup 1.0000x reference)
import jax
import jax.numpy as jnp
from jax.experimental import pallas as pl


def kernel(x, table, bias):
    raise NotImplementedError("write your pallas kernel here")


# speedup vs baseline: 28.9232x; 28.9232x over previous
import functools

import jax
import jax.numpy as jnp
from jax import lax
from jax.experimental import pallas as pl
from jax.experimental.pallas import tpu as pltpu
from jax.experimental.pallas import tpu_sc as plsc

B = 16384
N_FIELDS = 39
CVL = 13
N_IDX = N_FIELDS - CVL
N_UNW = N_IDX - CVL
OUT_DIM = 16
VOCAB = 100000

NC = 2
NS = 16
NW = NC * NS
CHUNK = 256
NCHUNK = B // CHUNK
CH_PER_W = NCHUNK // NW
IG = 128
GPC = CHUNK // IG

_MESH = plsc.VectorSubcoreMesh(
    core_axis_name="c", subcore_axis_name="s", num_cores=NC, num_subcores=NS
)


@functools.partial(
    pl.kernel,
    out_type=jax.ShapeDtypeStruct((B, OUT_DIM), jnp.float32),
    mesh=_MESH,
    compiler_params=pltpu.CompilerParams(use_tc_tiling_on_sc=False),
    scratch_types=[
        pltpu.VMEM((N_IDX, GPC, IG), jnp.int32),
        pltpu.VMEM((CVL, CHUNK), jnp.float32),
        pltpu.VMEM((CVL, CHUNK, OUT_DIM), jnp.float32),
        pltpu.VMEM((CHUNK, OUT_DIM), jnp.float32),
        pltpu.VMEM((OUT_DIM,), jnp.float32),
        pltpu.SemaphoreType.DMA,
    ],
)
def _fl_kernel(table_h, idx_h, cont_h, bias_h, out_h,
               idx_v, cont_v, emb_v, acc_v, bias_v, sem):
    wid = lax.axis_index("s") * NC + lax.axis_index("c")
    pltpu.sync_copy(bias_h, bias_v)

    def do_chunk(ci, carry):
        c = wid * CH_PER_W + ci
        pltpu.sync_copy(idx_h.at[c], idx_v)
        pltpu.sync_copy(cont_h.at[c], cont_v)

        cps = []
        for f in range(CVL):
            for j in range(GPC):
                cps.append(pltpu.async_copy(
                    table_h.at[idx_v.at[N_UNW + f, j]],
                    emb_v.at[f, pl.ds(j * IG, IG)], sem))
        for cp in cps:
            cp.wait()

        bias_vec = bias_v[...]

        def group_body(g, _):
            wvecs = [cont_v[f, pl.ds(g * 16, 16)] for f in range(CVL)]
            for l in range(16):
                r = g * 16 + l
                acc = bias_vec
                for f in range(CVL):
                    acc = acc + emb_v[f, r, :] * wvecs[f][l]
                acc_v[r, :] = acc
            return _

        lax.fori_loop(0, CHUNK // 16, group_body, None)

        cps = []
        for f in range(N_UNW):
            for j in range(GPC):
                cps.append(pltpu.async_copy(
                    table_h.at[idx_v.at[f, j]],
                    acc_v.at[pl.ds(j * IG, IG)], sem, add=True))
        for cp in cps:
            cp.wait()

        pltpu.sync_copy(acc_v, out_h.at[pl.ds(c * CHUNK, CHUNK)])
        return carry

    lax.fori_loop(0, CH_PER_W, do_chunk, None)


def kernel(x, table, bias):
    idx = x[:, :N_IDX].astype(jnp.int32)
    cont = x[:, N_IDX:]
    idx_prep = idx.T.reshape(N_IDX, NCHUNK, GPC, IG).transpose(1, 0, 2, 3)
    cont_prep = cont.T.reshape(CVL, NCHUNK, CHUNK).transpose(1, 0, 2)
    return _fl_kernel(table, idx_prep, cont_prep, bias)
